# baseline (speedup 1.0000x reference)
import functools

import numpy as np
import jax
import jax.numpy as jnp
from jax import lax
from jax.experimental import pallas as pl
from jax.experimental.pallas import tpu as pltpu
from jax.experimental.pallas import tpu_sc as plsc

NC = 37
NCP = 40
VEC = 142
NROW = 51
NF = 71
BB = 64
PR = 56
PV = 256

NWK = 32
RPW = 10 * 4096 // NWK
RCH = 128

_S = np.zeros((23, NF), np.float32)
_R = np.zeros((NF,), np.float32)
_MSK = np.zeros((NF,), np.float32)
for _k in range(3):
    for _r in range(3):
        _j = 1 + 3 * _k + _r
        _S[1 + _k, _j] = 1.0
        _R[_j] = _r
        _MSK[_j] = 1.0
for _k in range(7):
    for _r in range(7):
        _j = 10 + 7 * _k + _r
        _S[4 + _k, _j] = 1.0
        _R[_j] = _r
        _MSK[_j] = 1.0
for _j in range(12):
    _S[11 + _j, 59 + _j] = 1.0
_E0 = np.zeros((NF,), np.float32)
_E0[0] = 1.0


def _body(ch_ref, sc_ref, tr_ref, s_ref, aux_ref, m_ref, w1_ref, b1_ref,
          w2_ref, b2_ref, out_ref):
    f32 = jnp.float32
    ch2 = ch_ref[...]
    G = jnp.dot(ch2, s_ref[...], preferred_element_type=f32)
    msk = aux_ref[1, :][None, :] != 0.0
    F = jnp.where(msk, (G == aux_ref[0, :][None, :]).astype(f32), G) + aux_ref[2, :][None, :]
    rows = jnp.dot(F, m_ref[...], preferred_element_type=f32)
    out_ref[:, 0:NC, 0:VEC] = rows.reshape(BB, NCP, VEC)[:, 0:NC, :]

    x = jnp.clip(sc_ref[...], 0.0, 200.0) * ((VEC - 1) / 200.0)
    low = jnp.floor(x)
    frac = (x - low)[..., None]
    lowb = low[..., None]
    high = jnp.minimum(lowb + 1.0, float(VEC - 1))
    p = lax.broadcasted_iota(jnp.int32, (BB, 3, VEC), 2).astype(f32)
    enc = jnp.where(p == lowb, 1.0 - frac, 0.0) + jnp.where(p == high, frac, 0.0)
    out_ref[:, NC:NC + 3, 0:VEC] = enc

    h = jnp.maximum(
        jnp.dot(tr_ref[...], w1_ref[...], preferred_element_type=f32) + b1_ref[0, :], 0.0)
    y = jnp.dot(h, w2_ref[...], preferred_element_type=f32) + b2_ref[0, :]
    out_ref[:, NC + 13:NROW, 0:VEC] = y[:, None, :]


def _sc_bench(out2d, idxs, dests, btab, tok_out, idx_v, dest_v, rows_v, tok_v,
              sem1, sem2):
    wid = lax.axis_index("s") * 2 + lax.axis_index("c")
    for t in range(RPW // RCH):
        off = wid * RPW + t * RCH
        pltpu.sync_copy(idxs.at[pl.ds(off, RCH)], idx_v)
        pltpu.sync_copy(dests.at[pl.ds(off, RCH)], dest_v)
        pltpu.async_copy(btab.at[idx_v], rows_v, sem1).wait()
        pltpu.async_copy(rows_v, out2d.at[dest_v], sem2).wait()
    tok_v[...] = jnp.full((16,), wid, jnp.int32)
    pltpu.sync_copy(tok_v, tok_out.at[wid])


def kernel(champions, scalars, items, traits, champ_table, item_table, trait_table,
           bench_table, W1, b1, W2, b2):
    B = champions.shape[0]
    f32 = jnp.float32
    M = jnp.zeros((NF, VEC), f32)
    M = M.at[0, 0:30].set(champ_table[0])
    for k in range(3):
        M = M.at[1 + 3 * k:4 + 3 * k, 30 + 10 * k:40 + 10 * k].set(item_table)
    for k in range(7):
        M = M.at[10 + 7 * k:17 + 7 * k, 60 + 10 * k:70 + 10 * k].set(trait_table)
    M = M.at[59:NF, 130:VEC].set(jnp.eye(12, dtype=f32))

    ch40 = jnp.pad(champions, ((0, 0), (0, NCP - NC), (0, 0))).reshape(B * NCP, 23)

    full = lambda shp: pl.BlockSpec(shp, lambda i: (0,) * len(shp))
    padded = pl.pallas_call(
        _body,
        grid=(B // BB,),
        in_specs=[
            pl.BlockSpec((BB * NCP, 23), lambda i: (i, 0)),
            pl.BlockSpec((BB, 3), lambda i: (i, 0)),
            pl.BlockSpec((BB, 26), lambda i: (i, 0)),
            full((23, NF)), full((3, NF)), full((NF, VEC)),
            full((26, 26)), full((1, 26)), full((26, VEC)), full((1, VEC)),
        ],
        out_specs=pl.BlockSpec((BB, PR, PV), lambda i: (i, 0, 0)),
        out_shape=jax.ShapeDtypeStruct((B, PR, PV), jnp.float32),
    )(ch40, scalars, traits,
      jnp.asarray(_S), jnp.asarray(np.stack([_R, _MSK, _E0])), M,
      W1, b1.reshape(1, 26), W2, b2.reshape(1, VEC))

    padded2d = padded.reshape(B * PR, PV)
    idx_flat = items.astype(jnp.int32).reshape(B * 10)
    dest_flat = (jnp.arange(B, dtype=jnp.int32)[:, None] * PR + 40
                 + jnp.arange(10, dtype=jnp.int32)[None, :]).reshape(B * 10)
    btab256 = jnp.pad(bench_table, ((0, 0), (0, PV - VEC)))

    sc_call = functools.partial(
        pl.kernel,
        out_type=jax.ShapeDtypeStruct((NWK, 16), jnp.int32),
        mesh=plsc.VectorSubcoreMesh(core_axis_name="c", subcore_axis_name="s"),
        scratch_types=[
            pltpu.VMEM((RCH,), jnp.int32),
            pltpu.VMEM((RCH,), jnp.int32),
            pltpu.VMEM((RCH, PV), jnp.float32),
            pltpu.VMEM((16,), jnp.int32),
            pltpu.SemaphoreType.DMA,
            pltpu.SemaphoreType.DMA,
        ],
        compiler_params=pltpu.CompilerParams(has_side_effects=True),
    )(_sc_bench)
    token = sc_call(padded2d, idx_flat, dest_flat, btab256)

    p2, _ = lax.optimization_barrier((padded2d, token))
    return p2.reshape(B, PR, PV)[:, 0:NROW, 0:VEC]


# speedup vs baseline: 1.0004x; 1.0004x over previous
import functools

import numpy as np
import jax
import jax.numpy as jnp
from jax import lax
from jax.experimental import pallas as pl
from jax.experimental.pallas import tpu as pltpu
from jax.experimental.pallas import tpu_sc as plsc

NC = 37
NCP = 40
VEC = 142
NROW = 51
NF = 71
BB = 64
PR = 56
PV = 256

NWK = 32
RPW = 10 * 4096 // NWK
RCH = 128

_S = np.zeros((23, NF), np.float32)
_R = np.zeros((NF,), np.float32)
_MSK = np.zeros((NF,), np.float32)
for _k in range(3):
    for _r in range(3):
        _j = 1 + 3 * _k + _r
        _S[1 + _k, _j] = 1.0
        _R[_j] = _r
        _MSK[_j] = 1.0
for _k in range(7):
    for _r in range(7):
        _j = 10 + 7 * _k + _r
        _S[4 + _k, _j] = 1.0
        _R[_j] = _r
        _MSK[_j] = 1.0
for _j in range(12):
    _S[11 + _j, 59 + _j] = 1.0
_E0 = np.zeros((NF,), np.float32)
_E0[0] = 1.0


def _body(ch_ref, sc_ref, tr_ref, s_ref, aux_ref, m_ref, w1_ref, b1_ref,
          w2_ref, b2_ref, out_ref):
    f32 = jnp.float32
    ch2 = ch_ref[...]
    G = jnp.dot(ch2, s_ref[...], preferred_element_type=f32)
    msk = aux_ref[1, :][None, :] != 0.0
    F = jnp.where(msk, (G == aux_ref[0, :][None, :]).astype(f32), G) + aux_ref[2, :][None, :]
    rows = jnp.dot(F, m_ref[...], preferred_element_type=f32)
    out_ref[:, 0:NC, 0:VEC] = rows.reshape(BB, NCP, VEC)[:, 0:NC, :]

    x = jnp.clip(sc_ref[...], 0.0, 200.0) * ((VEC - 1) / 200.0)
    low = jnp.floor(x)
    frac = (x - low)[..., None]
    lowb = low[..., None]
    high = jnp.minimum(lowb + 1.0, float(VEC - 1))
    p = lax.broadcasted_iota(jnp.int32, (BB, 3, VEC), 2).astype(f32)
    enc = jnp.where(p == lowb, 1.0 - frac, 0.0) + jnp.where(p == high, frac, 0.0)
    out_ref[:, NC:NC + 3, 0:VEC] = enc

    h = jnp.maximum(
        jnp.dot(tr_ref[...], w1_ref[...], preferred_element_type=f32) + b1_ref[0, :], 0.0)
    y = jnp.dot(h, w2_ref[...], preferred_element_type=f32) + b2_ref[0, :]
    out_ref[:, NC + 13:NROW, 0:VEC] = y[:, None, :]


def _sc_bench(out2d, idxs, dests, btab, tok_out, idx_v, dest_v, rows_v0, rows_v1,
              tok_v, sem_g, sem_s):
    wid = lax.axis_index("s") * 2 + lax.axis_index("c")
    nt = RPW // RCH
    pltpu.sync_copy(idxs.at[wid], idx_v)
    pltpu.sync_copy(dests.at[wid], dest_v)
    bufs = (rows_v0, rows_v1)
    gathers = [None] * nt
    scatters = [None] * nt
    gathers[0] = pltpu.async_copy(btab.at[idx_v.at[0]], bufs[0], sem_g)
    for t in range(nt):
        gathers[t].wait()
        scatters[t] = pltpu.async_copy(bufs[t % 2], out2d.at[dest_v.at[t]], sem_s)
        if t + 1 < nt:
            if t >= 1:
                scatters[t - 1].wait()
            gathers[t + 1] = pltpu.async_copy(btab.at[idx_v.at[t + 1]],
                                              bufs[(t + 1) % 2], sem_g)
    if nt >= 2:
        scatters[nt - 2].wait()
    scatters[nt - 1].wait()
    tok_v[...] = jnp.full((16,), wid, jnp.int32)
    pltpu.sync_copy(tok_v, tok_out.at[wid])


def kernel(champions, scalars, items, traits, champ_table, item_table, trait_table,
           bench_table, W1, b1, W2, b2):
    B = champions.shape[0]
    f32 = jnp.float32
    M = jnp.zeros((NF, VEC), f32)
    M = M.at[0, 0:30].set(champ_table[0])
    for k in range(3):
        M = M.at[1 + 3 * k:4 + 3 * k, 30 + 10 * k:40 + 10 * k].set(item_table)
    for k in range(7):
        M = M.at[10 + 7 * k:17 + 7 * k, 60 + 10 * k:70 + 10 * k].set(trait_table)
    M = M.at[59:NF, 130:VEC].set(jnp.eye(12, dtype=f32))

    ch40 = jnp.pad(champions, ((0, 0), (0, NCP - NC), (0, 0))).reshape(B * NCP, 23)

    full = lambda shp: pl.BlockSpec(shp, lambda i: (0,) * len(shp))
    padded = pl.pallas_call(
        _body,
        grid=(B // BB,),
        in_specs=[
            pl.BlockSpec((BB * NCP, 23), lambda i: (i, 0)),
            pl.BlockSpec((BB, 3), lambda i: (i, 0)),
            pl.BlockSpec((BB, 26), lambda i: (i, 0)),
            full((23, NF)), full((3, NF)), full((NF, VEC)),
            full((26, 26)), full((1, 26)), full((26, VEC)), full((1, VEC)),
        ],
        out_specs=pl.BlockSpec((BB, PR, PV), lambda i: (i, 0, 0)),
        out_shape=jax.ShapeDtypeStruct((B, PR, PV), jnp.float32),
    )(ch40, scalars, traits,
      jnp.asarray(_S), jnp.asarray(np.stack([_R, _MSK, _E0])), M,
      W1, b1.reshape(1, 26), W2, b2.reshape(1, VEC))

    padded2d = padded.reshape(B * PR, PV)
    nt = RPW // RCH
    idx_3d = items.astype(jnp.int32).reshape(NWK, nt, RCH)
    dest_3d = (jnp.arange(B, dtype=jnp.int32)[:, None] * PR + 40
               + jnp.arange(10, dtype=jnp.int32)[None, :]).reshape(NWK, nt, RCH)
    btab256 = jnp.pad(bench_table, ((0, 0), (0, PV - VEC)))

    sc_call = functools.partial(
        pl.kernel,
        out_type=jax.ShapeDtypeStruct((NWK, 16), jnp.int32),
        mesh=plsc.VectorSubcoreMesh(core_axis_name="c", subcore_axis_name="s"),
        scratch_types=[
            pltpu.VMEM((nt, RCH), jnp.int32),
            pltpu.VMEM((nt, RCH), jnp.int32),
            pltpu.VMEM((RCH, PV), jnp.float32),
            pltpu.VMEM((RCH, PV), jnp.float32),
            pltpu.VMEM((16,), jnp.int32),
            pltpu.SemaphoreType.DMA,
            pltpu.SemaphoreType.DMA,
        ],
        compiler_params=pltpu.CompilerParams(has_side_effects=True),
    )(_sc_bench)
    token = sc_call(padded2d, idx_3d, dest_3d, btab256)

    p2, _ = lax.optimization_barrier((padded2d, token))
    return p2.reshape(B, PR, PV)[:, 0:NROW, 0:VEC]
